# baseline (speedup 1.0000x reference)
import functools

import jax
import jax.numpy as jnp
from jax import lax
from jax.experimental import pallas as pl
from jax.experimental.pallas import tpu as pltpu
from jax.experimental.pallas import tpu_sc as plsc

N = 10000
D = 128
E = 320000
EPS = 1e-5

NC = 2
NS = 16
NW = NC * NS
CH = 128
CPW = 80
EPW = CPW * CH
E_PAD = NW * EPW
NACC = 10240
RPT = NACC // NS
NZC = RPT // CH

BR = 400
NB = N // BR


_MESH = plsc.VectorSubcoreMesh(core_axis_name="c", subcore_axis_name="s")


def _deg_body(srcd_h, dstp_h, ones_h, zero_h, out_h,
              idx_v, ones_v, zeros_v, *accs):
    accs, () = accs[:6], accs[6:]
    cid = lax.axis_index("c")
    sid = lax.axis_index("s")
    wid = sid * NC + cid
    rows0 = sid * RPT
    pltpu.sync_copy(ones_h, ones_v)
    pltpu.sync_copy(zero_h, zeros_v)
    for k in range(6):
        for z in range(NZC):
            pltpu.sync_copy(zeros_v, accs[k].at[pl.ds(rows0 + z * CH, CH)])
    plsc.subcore_barrier()
    for k in range(6):
        src_side = k < 3
        rel = k % 3
        idx_h = srcd_h if src_side else dstp_h
        pltpu.sync_copy(idx_h.at[rel, wid], idx_v)

        def chunk(j, k=k):
            pltpu.sync_copy(ones_v, accs[k].at[idx_v.at[j]], add=True)

        lax.fori_loop(0, CPW, lambda j, _, chunk=chunk: (chunk(j), _)[1], None)
    plsc.subcore_barrier()
    for k in range(6):
        for z in range(NZC):
            sl = pl.ds(rows0 + z * CH, CH)
            pltpu.sync_copy(accs[k].at[sl], out_h.at[cid, k, sl])


_deg_kernel = functools.partial(
    pl.kernel,
    out_type=jax.ShapeDtypeStruct((NC, 6, NACC, 16), jnp.float32),
    mesh=_MESH,
    scratch_types=(
        [pltpu.VMEM((CPW, CH), jnp.int32),
         pltpu.VMEM((CH, 16), jnp.float32),
         pltpu.VMEM((CH, 16), jnp.float32)]
        + [pltpu.VMEM_SHARED((NACC, 16), jnp.float32) for _ in range(6)]
    ),
    compiler_params=pltpu.CompilerParams(use_tc_tiling_on_sc=False),
)(_deg_body)


DH = D // 2


WC = 2
NWIN = CPW // WC


def _spmm_body(glo_h, ghi_h, srcg_h, dstp_h, zero_h, out_h,
               src_v, dst_v, rows_a, rows_b, rows_c, zv, acc,
               sem_a, sem_b, sem_c, sem_z):
    cid = lax.axis_index("c")
    sid = lax.axis_index("s")
    wid = sid * NC + cid
    rows0 = sid * RPT
    bufs = (rows_a, rows_b, rows_c)
    sems = (sem_a, sem_b, sem_c)
    pltpu.sync_copy(zero_h, zv)
    for half, g_h in enumerate((glo_h, ghi_h)):
        for r in range(3):
            for z in range(NZC):
                pltpu.async_copy(zv, acc.at[pl.ds(rows0 + z * CH, CH)],
                                 sem_z)
            for z in range(NZC):
                pltpu.make_async_copy(
                    zv, acc.at[pl.ds(rows0 + z * CH, CH)], sem_z).wait()
            pltpu.sync_copy(srcg_h.at[r, wid], src_v)
            pltpu.sync_copy(dstp_h.at[r, wid], dst_v)
            plsc.subcore_barrier()

            def start(w, b, g_h=g_h):
                for k in range(WC):
                    pltpu.async_copy(g_h.at[src_v.at[w * WC + k]],
                                     bufs[b].at[k], sems[b])

            def drain(w, b, g_h=g_h):
                for k in range(WC):
                    pltpu.make_async_copy(
                        g_h.at[src_v.at[w * WC + k]], bufs[b].at[k],
                        sems[b]).wait()
                for k in range(WC):
                    pltpu.sync_copy(bufs[b].at[k],
                                    acc.at[dst_v.at[w * WC + k]], add=True)

            start(0, 0)
            start(1, 1)

            def lbody(i, _):
                base = 6 * i
                for t in range(6):
                    drain(base + t, t % 3)
                    start(base + t + 2, (t + 2) % 3)
                return _

            nfull = (NWIN - 2) // 6
            lax.fori_loop(0, nfull, lbody, None)
            w0 = 6 * nfull
            for w in range(w0, NWIN - 2):
                drain(w, w % 3)
                start(w + 2, (w + 2) % 3)
            drain(NWIN - 2, (NWIN - 2) % 3)
            drain(NWIN - 1, (NWIN - 1) % 3)
            plsc.subcore_barrier()
            for z in range(NZC):
                sl = pl.ds(rows0 + z * CH, CH)
                pltpu.async_copy(acc.at[sl], out_h.at[cid, r, half, sl],
                                 sem_z)
            for z in range(NZC):
                sl = pl.ds(rows0 + z * CH, CH)
                pltpu.make_async_copy(acc.at[sl], out_h.at[cid, r, half, sl],
                                      sem_z).wait()
            plsc.subcore_barrier()


_spmm_kernel = functools.partial(
    pl.kernel,
    out_type=jax.ShapeDtypeStruct((NC, 3, 2, NACC, DH), jnp.float32),
    mesh=_MESH,
    scratch_types=[
        pltpu.VMEM((CPW, CH), jnp.int32),
        pltpu.VMEM((CPW, CH), jnp.int32),
        pltpu.VMEM((WC, CH, DH), jnp.float32),
        pltpu.VMEM((WC, CH, DH), jnp.float32),
        pltpu.VMEM((WC, CH, DH), jnp.float32),
        pltpu.VMEM((CH, DH), jnp.float32),
        pltpu.VMEM_SHARED((NACC, DH), jnp.float32),
        pltpu.SemaphoreType.DMA,
        pltpu.SemaphoreType.DMA,
        pltpu.SemaphoreType.DMA,
        pltpu.SemaphoreType.DMA,
    ],
    compiler_params=pltpu.CompilerParams(use_tc_tiling_on_sc=False),
)(_spmm_body)


_PREC = lax.Precision.HIGHEST


def _rsqrt_body(deg_ref, s_ref):
    d = deg_ref[0] + deg_ref[1]
    s_ref[...] = lax.rsqrt(jnp.maximum(d, 1.0))


def _tc_rsqrt(degs):
    return pl.pallas_call(
        _rsqrt_body,
        grid=(10,),
        in_specs=[pl.BlockSpec((2, 6, NACC // 10, 16), lambda i: (0, 0, i, 0))],
        out_specs=pl.BlockSpec((6, NACC // 10, 16), lambda i: (0, i, 0)),
        out_shape=jax.ShapeDtypeStruct((6, NACC, 16), jnp.float32),
    )(degs)


def _scale_mm_body(h_ref, s_ref, w_ref, glo_ref, ghi_ref):
    hs = h_ref[...] * s_ref[0, :, 0:1]
    g = jnp.dot(hs, w_ref[0], precision=_PREC,
                preferred_element_type=jnp.float32)
    glo_ref[0] = g[:, :DH]
    ghi_ref[0] = g[:, DH:]


def _tc_scale_mm(h, s6, w3):
    return pl.pallas_call(
        _scale_mm_body,
        grid=(3, NB),
        in_specs=[
            pl.BlockSpec((BR, D), lambda r, i: (i, 0)),
            pl.BlockSpec((1, BR, 16), lambda r, i: (r, i, 0)),
            pl.BlockSpec((1, D, D), lambda r, i: (r, 0, 0)),
        ],
        out_specs=[
            pl.BlockSpec((1, BR, DH), lambda r, i: (r, i, 0)),
            pl.BlockSpec((1, BR, DH), lambda r, i: (r, i, 0)),
        ],
        out_shape=[
            jax.ShapeDtypeStruct((3, N, DH), jnp.float32),
            jax.ShapeDtypeStruct((3, N, DH), jnp.float32),
        ],
    )(h, s6, w3)


def _dense_body(agg_ref, s_ref, bsum_ref, fcw_ref, fcb_ref, v_ref, st_ref):
    i = pl.program_id(0)
    a = agg_ref[0] + agg_ref[1]
    t = jnp.concatenate(
        [a[0, 0] * s_ref[0, :, 0:1]
         + a[1, 0] * s_ref[1, :, 0:1]
         + a[2, 0] * s_ref[2, :, 0:1],
         a[0, 1] * s_ref[0, :, 0:1]
         + a[1, 1] * s_ref[1, :, 0:1]
         + a[2, 1] * s_ref[2, :, 0:1]], axis=-1)
    t = t + bsum_ref[...]
    u = jnp.dot(t, fcw_ref[...], precision=_PREC,
                preferred_element_type=jnp.float32) + fcb_ref[...]
    v = jnp.maximum(u, 0.0)
    v_ref[...] = v

    @pl.when(i == 0)
    def _():
        st_ref[...] = jnp.zeros_like(st_ref)

    st_ref[0:1, :] += jnp.sum(v, axis=0, keepdims=True)
    st_ref[1:2, :] += jnp.sum(v * v, axis=0, keepdims=True)


def _tc_dense(agg, s6, bsum, fcw, fcb):
    return pl.pallas_call(
        _dense_body,
        grid=(NB,),
        in_specs=[
            pl.BlockSpec((NC, 3, 2, BR, DH), lambda i: (0, 0, 0, i, 0)),
            pl.BlockSpec((3, BR, 16), lambda i: (1, i, 0)),
            pl.BlockSpec((1, D), lambda i: (0, 0)),
            pl.BlockSpec((D, D), lambda i: (0, 0)),
            pl.BlockSpec((1, D), lambda i: (0, 0)),
        ],
        out_specs=[
            pl.BlockSpec((BR, D), lambda i: (i, 0)),
            pl.BlockSpec((8, D), lambda i: (0, 0)),
        ],
        out_shape=[
            jax.ShapeDtypeStruct((N, D), jnp.float32),
            jax.ShapeDtypeStruct((8, D), jnp.float32),
        ],
    )(agg, s6, bsum, fcw, fcb)


def _bn_scale_body(v_ref, st_ref, g_ref, b_ref, s_ref, w_ref, olo_ref, ohi_ref):
    mu = st_ref[0:1, :] * (1.0 / N)
    ex2 = st_ref[1:2, :] * (1.0 / N)
    var = ex2 - mu * mu
    inv = lax.rsqrt(var + EPS)
    y = (v_ref[...] - mu) * (inv * g_ref[...]) + b_ref[...]
    ys = y * s_ref[0, :, 0:1]
    o = jnp.dot(ys, w_ref[0], precision=_PREC,
                preferred_element_type=jnp.float32)
    olo_ref[0] = o[:, :DH]
    ohi_ref[0] = o[:, DH:]


def _tc_bn_scale_mm(v, stats, gamma, beta, s6, w3):
    return pl.pallas_call(
        _bn_scale_body,
        grid=(3, NB),
        in_specs=[
            pl.BlockSpec((BR, D), lambda r, i: (i, 0)),
            pl.BlockSpec((8, D), lambda r, i: (0, 0)),
            pl.BlockSpec((1, D), lambda r, i: (0, 0)),
            pl.BlockSpec((1, D), lambda r, i: (0, 0)),
            pl.BlockSpec((1, BR, 16), lambda r, i: (r, i, 0)),
            pl.BlockSpec((1, D, D), lambda r, i: (r, 0, 0)),
        ],
        out_specs=[
            pl.BlockSpec((1, BR, DH), lambda r, i: (r, i, 0)),
            pl.BlockSpec((1, BR, DH), lambda r, i: (r, i, 0)),
        ],
        out_shape=[
            jax.ShapeDtypeStruct((3, N, DH), jnp.float32),
            jax.ShapeDtypeStruct((3, N, DH), jnp.float32),
        ],
    )(v, stats, gamma, beta, s6, w3)


def _bn_final_body(v_ref, st_ref, g_ref, b_ref, o_ref):
    mu = st_ref[0:1, :] * (1.0 / N)
    ex2 = st_ref[1:2, :] * (1.0 / N)
    var = ex2 - mu * mu
    inv = lax.rsqrt(var + EPS)
    o_ref[...] = (v_ref[...] - mu) * (inv * g_ref[...]) + b_ref[...]


def _tc_bn_final(v, stats, gamma, beta):
    return pl.pallas_call(
        _bn_final_body,
        grid=(NB,),
        in_specs=[
            pl.BlockSpec((BR, D), lambda i: (i, 0)),
            pl.BlockSpec((8, D), lambda i: (0, 0)),
            pl.BlockSpec((1, D), lambda i: (0, 0)),
            pl.BlockSpec((1, D), lambda i: (0, 0)),
        ],
        out_specs=pl.BlockSpec((BR, D), lambda i: (i, 0)),
        out_shape=jax.ShapeDtypeStruct((N, D), jnp.float32),
    )(v, stats, gamma, beta)


def _pad_idx(v, fill):
    return jnp.concatenate([v.astype(jnp.int32), fill]).reshape(NW, CPW, CH)


def kernel(x, ei_seq, ei_knn, ei_dis,
           W_seq0, b_seq0, W_knn0, b_knn0, W_dis0, b_dis0, fcW0, fcb0, bng0, bnb0,
           W_seq1, b_seq1, W_knn1, b_knn1, W_dis1, b_dis1, fcW1, fcb1, bng1, bnb1):
    eis = [ei_seq, ei_knn, ei_dis]
    npad = E_PAD - E
    ar = jnp.arange(npad, dtype=jnp.int32)
    fill_dummy = N + (ar % (NACC - N))
    fill_real = ar % N

    srcg3 = jnp.stack([_pad_idx(e[0], fill_real) + r * N
                       for r, e in enumerate(eis)])
    srcd3 = jnp.stack([_pad_idx(e[0], fill_dummy) for e in eis])
    dstp3 = jnp.stack([_pad_idx(e[1], fill_dummy) for e in eis])

    ones16 = jnp.ones((CH, 16), jnp.float32)
    zero16 = jnp.zeros((CH, 16), jnp.float32)
    zeroD = jnp.zeros((CH, DH), jnp.float32)

    degs = _deg_kernel(srcd3, dstp3, ones16, zero16)
    s6 = _tc_rsqrt(degs)

    w3_0 = jnp.stack([W_seq0, W_knn0, W_dis0])
    w3_1 = jnp.stack([W_seq1, W_knn1, W_dis1])
    bsum0 = (b_seq0 + b_knn0 + b_dis0).reshape(1, D)
    bsum1 = (b_seq1 + b_knn1 + b_dis1).reshape(1, D)

    g0lo, g0hi = _tc_scale_mm(x, s6, w3_0)
    agg0 = _spmm_kernel(g0lo.reshape(3 * N, DH), g0hi.reshape(3 * N, DH),
                        srcg3, dstp3, zeroD)
    v0, st0 = _tc_dense(agg0, s6, bsum0, fcW0, fcb0.reshape(1, D))

    g1lo, g1hi = _tc_bn_scale_mm(v0, st0, bng0.reshape(1, D), bnb0.reshape(1, D),
                                 s6, w3_1)
    agg1 = _spmm_kernel(g1lo.reshape(3 * N, DH), g1hi.reshape(3 * N, DH),
                        srcg3, dstp3, zeroD)
    v1, st1 = _tc_dense(agg1, s6, bsum1, fcW1, fcb1.reshape(1, D))

    return _tc_bn_final(v1, st1, bng1.reshape(1, D), bnb1.reshape(1, D))


# speedup vs baseline: 1.0397x; 1.0397x over previous
import functools

import jax
import jax.numpy as jnp
from jax import lax
from jax.experimental import pallas as pl
from jax.experimental.pallas import tpu as pltpu
from jax.experimental.pallas import tpu_sc as plsc

N = 10000
D = 128
E = 320000
EPS = 1e-5

NC = 2
NS = 16
NW = NC * NS
CH = 128
CPW = 80
EPW = CPW * CH
E_PAD = NW * EPW
NACC = 10240
RPT = NACC // NS
NZC = RPT // CH

BR = 400
NB = N // BR


_MESH = plsc.VectorSubcoreMesh(core_axis_name="c", subcore_axis_name="s")


def _deg_body(srcd_h, dstp_h, ones_h, zero_h, out_h,
              idx_v, ones_v, zeros_v, sem_d, *accs):
    accs, () = accs[:6], accs[6:]
    cid = lax.axis_index("c")
    sid = lax.axis_index("s")
    wid = sid * NC + cid
    rows0 = sid * RPT
    pltpu.sync_copy(ones_h, ones_v)
    pltpu.sync_copy(zero_h, zeros_v)
    for k in range(6):
        for z in range(NZC):
            pltpu.sync_copy(zeros_v, accs[k].at[pl.ds(rows0 + z * CH, CH)])
    plsc.subcore_barrier()
    DW = 10
    for k in range(6):
        src_side = k < 3
        rel = k % 3
        idx_h = srcd_h if src_side else dstp_h
        pltpu.sync_copy(idx_h.at[rel, wid], idx_v)

        def wbody(w, _, k=k):
            for j in range(DW):
                pltpu.async_copy(ones_v, accs[k].at[idx_v.at[w * DW + j]],
                                 sem_d, add=True)
            for j in range(DW):
                pltpu.make_async_copy(ones_v,
                                      accs[k].at[idx_v.at[w * DW + j]],
                                      sem_d).wait()
            return _

        lax.fori_loop(0, CPW // DW, wbody, None)
    plsc.subcore_barrier()
    for k in range(6):
        for z in range(NZC):
            sl = pl.ds(rows0 + z * CH, CH)
            pltpu.sync_copy(accs[k].at[sl], out_h.at[cid, k, sl])


_deg_kernel = functools.partial(
    pl.kernel,
    out_type=jax.ShapeDtypeStruct((NC, 6, NACC, 16), jnp.float32),
    mesh=_MESH,
    scratch_types=(
        [pltpu.VMEM((CPW, CH), jnp.int32),
         pltpu.VMEM((CH, 16), jnp.float32),
         pltpu.VMEM((CH, 16), jnp.float32),
         pltpu.SemaphoreType.DMA]
        + [pltpu.VMEM_SHARED((NACC, 16), jnp.float32) for _ in range(6)]
    ),
    compiler_params=pltpu.CompilerParams(use_tc_tiling_on_sc=False),
)(_deg_body)


DH = D // 2


WC = 2
NWIN = CPW // WC


def _spmm_body(glo_h, ghi_h, srcg_h, dstp_h, zero_h, out_h,
               src_v, dst_v, rows_a, rows_b, rows_c, zv, acc,
               sem_a, sem_b, sem_c, sem_z):
    cid = lax.axis_index("c")
    sid = lax.axis_index("s")
    wid = sid * NC + cid
    rows0 = sid * RPT
    bufs = (rows_a, rows_b, rows_c)
    sems = (sem_a, sem_b, sem_c)
    pltpu.sync_copy(zero_h, zv)
    for half, g_h in enumerate((glo_h, ghi_h)):
        for r in range(3):
            for z in range(NZC):
                pltpu.async_copy(zv, acc.at[pl.ds(rows0 + z * CH, CH)],
                                 sem_z)
            for z in range(NZC):
                pltpu.make_async_copy(
                    zv, acc.at[pl.ds(rows0 + z * CH, CH)], sem_z).wait()
            pltpu.sync_copy(srcg_h.at[r, wid], src_v)
            pltpu.sync_copy(dstp_h.at[r, wid], dst_v)
            plsc.subcore_barrier()

            def start(w, b, g_h=g_h):
                for k in range(WC):
                    pltpu.async_copy(g_h.at[src_v.at[w * WC + k]],
                                     bufs[b].at[k], sems[b])

            def drain(w, b, g_h=g_h):
                for k in range(WC):
                    pltpu.make_async_copy(
                        g_h.at[src_v.at[w * WC + k]], bufs[b].at[k],
                        sems[b]).wait()
                for k in range(WC):
                    pltpu.sync_copy(bufs[b].at[k],
                                    acc.at[dst_v.at[w * WC + k]], add=True)

            start(0, 0)
            start(1, 1)

            def lbody(i, _):
                base = 6 * i
                for t in range(6):
                    drain(base + t, t % 3)
                    start(base + t + 2, (t + 2) % 3)
                return _

            nfull = (NWIN - 2) // 6
            lax.fori_loop(0, nfull, lbody, None)
            w0 = 6 * nfull
            for w in range(w0, NWIN - 2):
                drain(w, w % 3)
                start(w + 2, (w + 2) % 3)
            drain(NWIN - 2, (NWIN - 2) % 3)
            drain(NWIN - 1, (NWIN - 1) % 3)
            plsc.subcore_barrier()
            for z in range(NZC):
                sl = pl.ds(rows0 + z * CH, CH)
                pltpu.async_copy(acc.at[sl], out_h.at[cid, r, half, sl],
                                 sem_z)
            for z in range(NZC):
                sl = pl.ds(rows0 + z * CH, CH)
                pltpu.make_async_copy(acc.at[sl], out_h.at[cid, r, half, sl],
                                      sem_z).wait()
            plsc.subcore_barrier()


_spmm_kernel = functools.partial(
    pl.kernel,
    out_type=jax.ShapeDtypeStruct((NC, 3, 2, NACC, DH), jnp.float32),
    mesh=_MESH,
    scratch_types=[
        pltpu.VMEM((CPW, CH), jnp.int32),
        pltpu.VMEM((CPW, CH), jnp.int32),
        pltpu.VMEM((WC, CH, DH), jnp.float32),
        pltpu.VMEM((WC, CH, DH), jnp.float32),
        pltpu.VMEM((WC, CH, DH), jnp.float32),
        pltpu.VMEM((CH, DH), jnp.float32),
        pltpu.VMEM_SHARED((NACC, DH), jnp.float32),
        pltpu.SemaphoreType.DMA,
        pltpu.SemaphoreType.DMA,
        pltpu.SemaphoreType.DMA,
        pltpu.SemaphoreType.DMA,
    ],
    compiler_params=pltpu.CompilerParams(use_tc_tiling_on_sc=False),
)(_spmm_body)


_PREC = lax.Precision.HIGHEST


def _scale_mm_body(h_ref, dego_ref, degi_ref, w_ref,
                   glo_ref, ghi_ref, sout_ref, sin_ref):
    so = lax.rsqrt(jnp.maximum(dego_ref[0, 0] + dego_ref[1, 0], 1.0))
    si = lax.rsqrt(jnp.maximum(degi_ref[0, 0] + degi_ref[1, 0], 1.0))
    sout_ref[0] = so
    sin_ref[0] = si
    hs = h_ref[...] * so[:, 0:1]
    g = jnp.dot(hs, w_ref[0], precision=_PREC,
                preferred_element_type=jnp.float32)
    glo_ref[0] = g[:, :DH]
    ghi_ref[0] = g[:, DH:]


def _tc_scale_mm(h, degs, w3):
    return pl.pallas_call(
        _scale_mm_body,
        grid=(3, NB),
        in_specs=[
            pl.BlockSpec((BR, D), lambda r, i: (i, 0)),
            pl.BlockSpec((NC, 1, BR, 16), lambda r, i: (0, r, i, 0)),
            pl.BlockSpec((NC, 1, BR, 16), lambda r, i: (0, r + 3, i, 0)),
            pl.BlockSpec((1, D, D), lambda r, i: (r, 0, 0)),
        ],
        out_specs=[
            pl.BlockSpec((1, BR, DH), lambda r, i: (r, i, 0)),
            pl.BlockSpec((1, BR, DH), lambda r, i: (r, i, 0)),
            pl.BlockSpec((1, BR, 16), lambda r, i: (r, i, 0)),
            pl.BlockSpec((1, BR, 16), lambda r, i: (r, i, 0)),
        ],
        out_shape=[
            jax.ShapeDtypeStruct((3, N, DH), jnp.float32),
            jax.ShapeDtypeStruct((3, N, DH), jnp.float32),
            jax.ShapeDtypeStruct((3, N, 16), jnp.float32),
            jax.ShapeDtypeStruct((3, N, 16), jnp.float32),
        ],
    )(h, degs, degs, w3)


def _dense_body(agg_ref, s_ref, bsum_ref, fcw_ref, fcb_ref, v_ref, st_ref):
    i = pl.program_id(0)
    a = agg_ref[0] + agg_ref[1]
    t = jnp.concatenate(
        [a[0, 0] * s_ref[0, :, 0:1]
         + a[1, 0] * s_ref[1, :, 0:1]
         + a[2, 0] * s_ref[2, :, 0:1],
         a[0, 1] * s_ref[0, :, 0:1]
         + a[1, 1] * s_ref[1, :, 0:1]
         + a[2, 1] * s_ref[2, :, 0:1]], axis=-1)
    t = t + bsum_ref[...]
    u = jnp.dot(t, fcw_ref[...], precision=_PREC,
                preferred_element_type=jnp.float32) + fcb_ref[...]
    v = jnp.maximum(u, 0.0)
    v_ref[...] = v

    @pl.when(i == 0)
    def _():
        st_ref[...] = jnp.zeros_like(st_ref)

    st_ref[0:1, :] += jnp.sum(v, axis=0, keepdims=True)
    st_ref[1:2, :] += jnp.sum(v * v, axis=0, keepdims=True)


def _tc_dense(agg, sin, bsum, fcw, fcb):
    return pl.pallas_call(
        _dense_body,
        grid=(NB,),
        in_specs=[
            pl.BlockSpec((NC, 3, 2, BR, DH), lambda i: (0, 0, 0, i, 0)),
            pl.BlockSpec((3, BR, 16), lambda i: (0, i, 0)),
            pl.BlockSpec((1, D), lambda i: (0, 0)),
            pl.BlockSpec((D, D), lambda i: (0, 0)),
            pl.BlockSpec((1, D), lambda i: (0, 0)),
        ],
        out_specs=[
            pl.BlockSpec((BR, D), lambda i: (i, 0)),
            pl.BlockSpec((8, D), lambda i: (0, 0)),
        ],
        out_shape=[
            jax.ShapeDtypeStruct((N, D), jnp.float32),
            jax.ShapeDtypeStruct((8, D), jnp.float32),
        ],
    )(agg, sin, bsum, fcw, fcb)


def _bn_scale_body(v_ref, st_ref, g_ref, b_ref, s_ref, w_ref, olo_ref, ohi_ref):
    mu = st_ref[0:1, :] * (1.0 / N)
    ex2 = st_ref[1:2, :] * (1.0 / N)
    var = ex2 - mu * mu
    inv = lax.rsqrt(var + EPS)
    y = (v_ref[...] - mu) * (inv * g_ref[...]) + b_ref[...]
    ys = y * s_ref[0, :, 0:1]
    o = jnp.dot(ys, w_ref[0], precision=_PREC,
                preferred_element_type=jnp.float32)
    olo_ref[0] = o[:, :DH]
    ohi_ref[0] = o[:, DH:]


def _tc_bn_scale_mm(v, stats, gamma, beta, sout, w3):
    return pl.pallas_call(
        _bn_scale_body,
        grid=(3, NB),
        in_specs=[
            pl.BlockSpec((BR, D), lambda r, i: (i, 0)),
            pl.BlockSpec((8, D), lambda r, i: (0, 0)),
            pl.BlockSpec((1, D), lambda r, i: (0, 0)),
            pl.BlockSpec((1, D), lambda r, i: (0, 0)),
            pl.BlockSpec((1, BR, 16), lambda r, i: (r, i, 0)),
            pl.BlockSpec((1, D, D), lambda r, i: (r, 0, 0)),
        ],
        out_specs=[
            pl.BlockSpec((1, BR, DH), lambda r, i: (r, i, 0)),
            pl.BlockSpec((1, BR, DH), lambda r, i: (r, i, 0)),
        ],
        out_shape=[
            jax.ShapeDtypeStruct((3, N, DH), jnp.float32),
            jax.ShapeDtypeStruct((3, N, DH), jnp.float32),
        ],
    )(v, stats, gamma, beta, sout, w3)


def _bn_final_body(v_ref, st_ref, g_ref, b_ref, o_ref):
    mu = st_ref[0:1, :] * (1.0 / N)
    ex2 = st_ref[1:2, :] * (1.0 / N)
    var = ex2 - mu * mu
    inv = lax.rsqrt(var + EPS)
    o_ref[...] = (v_ref[...] - mu) * (inv * g_ref[...]) + b_ref[...]


def _tc_bn_final(v, stats, gamma, beta):
    return pl.pallas_call(
        _bn_final_body,
        grid=(NB,),
        in_specs=[
            pl.BlockSpec((BR, D), lambda i: (i, 0)),
            pl.BlockSpec((8, D), lambda i: (0, 0)),
            pl.BlockSpec((1, D), lambda i: (0, 0)),
            pl.BlockSpec((1, D), lambda i: (0, 0)),
        ],
        out_specs=pl.BlockSpec((BR, D), lambda i: (i, 0)),
        out_shape=jax.ShapeDtypeStruct((N, D), jnp.float32),
    )(v, stats, gamma, beta)


def kernel(x, ei_seq, ei_knn, ei_dis,
           W_seq0, b_seq0, W_knn0, b_knn0, W_dis0, b_dis0, fcW0, fcb0, bng0, bnb0,
           W_seq1, b_seq1, W_knn1, b_knn1, W_dis1, b_dis1, fcW1, fcb1, bng1, bnb1):
    npad = E_PAD - E
    ar = jnp.arange(npad, dtype=jnp.int32)
    fill_dummy = N + (ar % (NACC - N))
    fill_real = ar % N

    ei3 = jnp.stack([ei_seq, ei_knn, ei_dis]).astype(jnp.int32)
    pad_d = jnp.broadcast_to(fill_dummy, (3, 2, npad))
    ei3p = jnp.concatenate([ei3, pad_d], axis=2).reshape(3, 2, NW, CPW, CH)
    srcd3 = ei3p[:, 0]
    dstp3 = ei3p[:, 1]
    srcg3 = (jnp.concatenate(
        [ei3[:, 0], jnp.broadcast_to(fill_real, (3, npad))], axis=1)
        + (jnp.arange(3, dtype=jnp.int32) * N)[:, None]).reshape(
            3, NW, CPW, CH)

    ones16 = jnp.ones((CH, 16), jnp.float32)
    zero16 = jnp.zeros((CH, 16), jnp.float32)
    zeroD = jnp.zeros((CH, DH), jnp.float32)

    degs = _deg_kernel(srcd3, dstp3, ones16, zero16)

    w3_0 = jnp.stack([W_seq0, W_knn0, W_dis0])
    w3_1 = jnp.stack([W_seq1, W_knn1, W_dis1])
    bsum0 = (b_seq0 + b_knn0 + b_dis0).reshape(1, D)
    bsum1 = (b_seq1 + b_knn1 + b_dis1).reshape(1, D)

    g0lo, g0hi, sout, sin = _tc_scale_mm(x, degs, w3_0)
    agg0 = _spmm_kernel(g0lo.reshape(3 * N, DH), g0hi.reshape(3 * N, DH),
                        srcg3, dstp3, zeroD)
    v0, st0 = _tc_dense(agg0, sin, bsum0, fcW0, fcb0.reshape(1, D))

    g1lo, g1hi = _tc_bn_scale_mm(v0, st0, bng0.reshape(1, D), bnb0.reshape(1, D),
                                 sout, w3_1)
    agg1 = _spmm_kernel(g1lo.reshape(3 * N, DH), g1hi.reshape(3 * N, DH),
                        srcg3, dstp3, zeroD)
    v1, st1 = _tc_dense(agg1, sin, bsum1, fcW1, fcb1.reshape(1, D))

    return _tc_bn_final(v1, st1, bng1.reshape(1, D), bnb1.reshape(1, D))
